# baseline (speedup 1.0000x reference)
import functools

import numpy as np
import jax
import jax.numpy as jnp
from jax import lax
from jax.experimental import pallas as pl
from jax.experimental.pallas import tpu as pltpu
from jax.experimental.pallas import tpu_sc as plsc

_N = 131072
_N_LEVELS = 16
_F = 2
_LOG2_T = 19
_T = 1 << _LOG2_T
_BASE_RES = 16
_MAX_RES = 512
_GROWTH = float(np.exp((np.log(_MAX_RES) - np.log(_BASE_RES)) / (_N_LEVELS - 1)))
_RES = [int(np.floor(_BASE_RES * (_GROWTH ** l))) for l in range(_N_LEVELS)]
_STRIDE = [r + 1 for r in _RES]
_IS_DENSE = [int((s ** 3) <= _T) for s in _STRIDE]
_P1 = np.uint32(2654435761).astype(np.int32)
_P2 = np.int32(805459861)

_NUM_VIEWS = 200
_VIEW_DIM = 32

_SC_CORES = 2
_SC_SUBCORES = 16
_NW = _SC_CORES * _SC_SUBCORES

_M = _N * 128
_MW = _M // _NW
_SLAB = 4096
_NSLAB = _MW // _SLAB
_NVW = _N // _NW
_VSLAB = 1024
_NVSLAB = _NVW // _VSLAB

_cf = np.zeros((8, 128), np.float32)
_ci = np.zeros((8, 128), np.int32)
for k in range(128):
    l = k % 16
    _cf[0, k] = float(_RES[l])
    _ci[0, k] = _STRIDE[l]
    _ci[1, k] = _STRIDE[l] * _STRIDE[l]
    _ci[2, k] = l * _T
    _ci[3, k] = _IS_DENSE[l]
_cf2 = np.zeros((8, 256), np.float32)
for j in range(256):
    l = (j % 32) // 2
    _cf2[0, j] = float(_RES[l])


def _idx_kernel(bmin_ref, bmax_ref, pts_ref, cf_ref, ci_ref, idx_ref):
    resf = cf_ref[0:1, :]
    stride = ci_ref[0:1, :]
    stride2 = ci_ref[1:2, :]
    lT = ci_ref[2:3, :]
    dense = ci_ref[3:4, :]
    lane = lax.broadcasted_iota(jnp.int32, (1, 128), 1)
    c = lane // 16
    dbits = [(c >> 2) & 1, (c >> 1) & 1, c & 1]
    coords = []
    for d in range(3):
        p = pts_ref[:, d:d + 1]
        p01 = jnp.clip((p - bmin_ref[d]) / (bmax_ref[d] - bmin_ref[d]), 0.0, 1.0)
        x = p01 * resf
        coords.append(jnp.floor(x).astype(jnp.int32) + dbits[d])
    cx, cy, cz = coords
    idx_d = cx + cy * stride + cz * stride2
    idx_h = (cx ^ (cy * _P1) ^ (cz * _P2)) & jnp.int32(_T - 1)
    idx_ref[...] = jnp.where(dense == 1, idx_d, idx_h) + lT


def _mlp_kernel(bmin_ref, bmax_ref, pts_ref, g_ref, ve_ref, w0_ref, w1_ref,
                w2_ref, wout_ref, cf2_ref, out_ref):
    resf = cf2_ref[0:1, :]
    lane = lax.broadcasted_iota(jnp.int32, (1, 256), 1)
    c = lane // 32
    dbits = [(c >> 2) & 1, (c >> 1) & 1, c & 1]
    wprod = None
    for d in range(3):
        p = pts_ref[:, d:d + 1]
        p01 = jnp.clip((p - bmin_ref[d]) / (bmax_ref[d] - bmin_ref[d]), 0.0, 1.0)
        x = p01 * resf
        frac = x - jnp.floor(x)
        wd = jnp.where(dbits[d] == 1, frac, 1.0 - frac)
        wprod = wd if wprod is None else wprod * wd
    prod = g_ref[...] * wprod
    feats = prod[:, 0:32]
    for cc in range(1, 8):
        feats = feats + prod[:, cc * 32:(cc + 1) * 32]
    comb = jnp.concatenate([feats, ve_ref[...]], axis=1)
    h = jnp.maximum(jnp.dot(comb, w0_ref[...], preferred_element_type=jnp.float32), 0.0)
    h = jnp.maximum(jnp.dot(h, w1_ref[...], preferred_element_type=jnp.float32), 0.0)
    h = jnp.maximum(jnp.dot(h, w2_ref[...], preferred_element_type=jnp.float32), 0.0)
    xi = jnp.dot(h, wout_ref[...], preferred_element_type=jnp.float32)
    lane6 = lax.broadcasted_iota(jnp.int32, (1, 6), 1)
    sx = bmax_ref[0] - bmin_ref[0]
    sy = bmax_ref[1] - bmin_ref[1]
    sz = bmax_ref[2] - bmin_ref[2]
    scale = jnp.where(lane6 == 3, sx,
                      jnp.where(lane6 == 4, sy,
                                jnp.where(lane6 == 5, sz, 1.0)))
    out_ref[...] = xi * scale


_B1 = 2048
_B3 = 1024


def _run_idx(bbox_min, bbox_max, pts, cf, ci):
    return pl.pallas_call(
        _idx_kernel,
        grid=(_N // _B1,),
        in_specs=[
            pl.BlockSpec(memory_space=pltpu.SMEM),
            pl.BlockSpec(memory_space=pltpu.SMEM),
            pl.BlockSpec((_B1, 3), lambda i: (i, 0)),
            pl.BlockSpec((8, 128), lambda i: (0, 0)),
            pl.BlockSpec((8, 128), lambda i: (0, 0)),
        ],
        out_specs=pl.BlockSpec((_B1, 128), lambda i: (i, 0)),
        out_shape=jax.ShapeDtypeStruct((_N, 128), jnp.int32),
    )(bbox_min, bbox_max, pts, cf, ci)


def _run_mlp(bbox_min, bbox_max, pts, g, ve, W0, W1, W2, Wout, cf2):
    return pl.pallas_call(
        _mlp_kernel,
        grid=(_N // _B3,),
        in_specs=[
            pl.BlockSpec(memory_space=pltpu.SMEM),
            pl.BlockSpec(memory_space=pltpu.SMEM),
            pl.BlockSpec((_B3, 3), lambda i: (i, 0)),
            pl.BlockSpec((_B3, 256), lambda i: (i, 0)),
            pl.BlockSpec((_B3, 32), lambda i: (i, 0)),
            pl.BlockSpec((64, 64), lambda i: (0, 0)),
            pl.BlockSpec((64, 64), lambda i: (0, 0)),
            pl.BlockSpec((64, 64), lambda i: (0, 0)),
            pl.BlockSpec((64, 6), lambda i: (0, 0)),
            pl.BlockSpec((8, 256), lambda i: (0, 0)),
        ],
        out_specs=pl.BlockSpec((_B3, 6), lambda i: (i, 0)),
        out_shape=jax.ShapeDtypeStruct((_N, 6), jnp.float32),
    )(bbox_min, bbox_max, pts, g, ve, W0, W1, W2, Wout, cf2)


def _sc_gather(tbl_flat, idx_flat, view_idx, view_table):
    mesh = plsc.VectorSubcoreMesh(core_axis_name="c", subcore_axis_name="s")

    @functools.partial(
        pl.kernel,
        out_type=[
            jax.ShapeDtypeStruct((_M, _F), jnp.float32),
            jax.ShapeDtypeStruct((_N, _VIEW_DIM), jnp.float32),
        ],
        mesh=mesh,
        scratch_types=[
            pltpu.VMEM((_SLAB,), jnp.int32),
            pltpu.VMEM((_SLAB, _F), jnp.float32),
            pltpu.VMEM((_VSLAB,), jnp.int32),
            pltpu.VMEM((_VSLAB, _VIEW_DIM), jnp.float32),
            pltpu.SemaphoreType.DMA,
        ],
        compiler_params=pltpu.CompilerParams(use_tc_tiling_on_sc=False),
    )
    def k(tbl_hbm, idx_hbm, vidx_hbm, vtbl_hbm, g_hbm, ve_hbm,
          idx_v, rows_v, vi_v, vrows_v, sem):
        wid = lax.axis_index("s") * _SC_CORES + lax.axis_index("c")
        base = wid * _MW

        @pl.loop(0, _NSLAB)
        def _(s):
            off = base + s * _SLAB
            pltpu.sync_copy(idx_hbm.at[pl.ds(off, _SLAB)], idx_v)
            pltpu.async_copy(tbl_hbm.at[idx_v], rows_v, sem).wait()
            pltpu.sync_copy(rows_v, g_hbm.at[pl.ds(off, _SLAB)])

        vbase = wid * _NVW

        @pl.loop(0, _NVSLAB)
        def _(s):
            off = vbase + s * _VSLAB
            pltpu.sync_copy(vidx_hbm.at[pl.ds(off, _VSLAB)], vi_v)
            pltpu.async_copy(vtbl_hbm.at[vi_v], vrows_v, sem).wait()
            pltpu.sync_copy(vrows_v, ve_hbm.at[pl.ds(off, _VSLAB)])

    return k(tbl_flat, idx_flat, view_idx, view_table)


def kernel(aligned_pts, view_idx, hash_tables, view_table, W0, W1, W2, Wout,
           bbox_min, bbox_max):
    cf = jnp.asarray(_cf)
    ci = jnp.asarray(_ci)
    cf2 = jnp.asarray(_cf2)
    tbl_flat = hash_tables.reshape(_N_LEVELS * _T, _F)
    idx = _run_idx(bbox_min, bbox_max, aligned_pts, cf, ci)
    g, ve = _sc_gather(tbl_flat, idx.reshape(_M), view_idx.astype(jnp.int32),
                       view_table)
    return _run_mlp(bbox_min, bbox_max, aligned_pts, g.reshape(_N, 256), ve,
                    W0, W1, W2, Wout, cf2)


# speedup vs baseline: 11.0132x; 11.0132x over previous
import functools

import numpy as np
import jax
import jax.numpy as jnp
from jax import lax
from jax.experimental import pallas as pl
from jax.experimental.pallas import tpu as pltpu
from jax.experimental.pallas import tpu_sc as plsc

_N = 131072
_N_LEVELS = 16
_F = 2
_LOG2_T = 19
_T = 1 << _LOG2_T
_BASE_RES = 16
_MAX_RES = 512
_GROWTH = float(np.exp((np.log(_MAX_RES) - np.log(_BASE_RES)) / (_N_LEVELS - 1)))
_RES = [int(np.floor(_BASE_RES * (_GROWTH ** l))) for l in range(_N_LEVELS)]
_STRIDE = [r + 1 for r in _RES]
_IS_DENSE = [int((s ** 3) <= _T) for s in _STRIDE]
_P1 = np.uint32(2654435761).astype(np.int32)
_P2 = np.int32(805459861)

_NUM_VIEWS = 200
_VIEW_DIM = 32

_SC_CORES = 2
_SC_SUBCORES = 16
_NW = _SC_CORES * _SC_SUBCORES

_M = _N * 128
_MW = _M // _NW
_SLAB = 8192
_NSLAB = _MW // _SLAB
_NVW = _N // _NW
_VSLAB = 1024
_NVSLAB = _NVW // _VSLAB

_cf = np.zeros((8, 128), np.float32)
_ci = np.zeros((8, 128), np.int32)
for k in range(128):
    l = k % 16
    _cf[0, k] = float(_RES[l])
    _ci[0, k] = _STRIDE[l]
    _ci[1, k] = _STRIDE[l] * _STRIDE[l]
    _ci[2, k] = l * _T
    _ci[3, k] = _IS_DENSE[l]


def _idx_kernel(bmin_ref, bmax_ref, pts_ref, cf_ref, ci_ref, idx_ref):
    resf = cf_ref[0:1, :]
    stride = ci_ref[0:1, :]
    stride2 = ci_ref[1:2, :]
    lT = ci_ref[2:3, :]
    dense = ci_ref[3:4, :]
    lane = lax.broadcasted_iota(jnp.int32, (1, 128), 1)
    c = lane // 16
    dbits = [(c >> 2) & 1, (c >> 1) & 1, c & 1]
    coords = []
    for d in range(3):
        p = pts_ref[:, d:d + 1]
        p01 = jnp.clip((p - bmin_ref[d]) / (bmax_ref[d] - bmin_ref[d]), 0.0, 1.0)
        x = p01 * resf
        coords.append(jnp.floor(x).astype(jnp.int32) + dbits[d])
    cx, cy, cz = coords
    idx_d = cx + cy * stride + cz * stride2
    idx_h = (cx ^ (cy * _P1) ^ (cz * _P2)) & jnp.int32(_T - 1)
    idx_ref[...] = jnp.where(dense == 1, idx_d, idx_h) + lT


def _mlp_kernel(bmin_ref, bmax_ref, pts_ref, g0_ref, g1_ref, ve_ref, w0_ref,
                w1_ref, w2_ref, wout_ref, cf_ref, ci_ref, out_ref):
    resf = cf_ref[0:1, :]
    lane = lax.broadcasted_iota(jnp.int32, (1, 128), 1)
    c = lane // 16
    dbits = [(c >> 2) & 1, (c >> 1) & 1, c & 1]
    w = None
    for d in range(3):
        p = pts_ref[:, d:d + 1]
        p01 = jnp.clip((p - bmin_ref[d]) / (bmax_ref[d] - bmin_ref[d]), 0.0, 1.0)
        x = p01 * resf
        frac = x - jnp.floor(x)
        wd = jnp.where(dbits[d] == 1, frac, 1.0 - frac)
        w = wd if w is None else w * wd
    s0 = g0_ref[...] * w
    s1 = g1_ref[...] * w
    f0 = s0[:, 0:16]
    f1 = s1[:, 0:16]
    for cc in range(1, 8):
        f0 = f0 + s0[:, cc * 16:(cc + 1) * 16]
        f1 = f1 + s1[:, cc * 16:(cc + 1) * 16]
    comb = jnp.concatenate([f0, f1, ve_ref[...]], axis=1)
    h = jnp.maximum(jnp.dot(comb, w0_ref[...], preferred_element_type=jnp.float32), 0.0)
    h = jnp.maximum(jnp.dot(h, w1_ref[...], preferred_element_type=jnp.float32), 0.0)
    h = jnp.maximum(jnp.dot(h, w2_ref[...], preferred_element_type=jnp.float32), 0.0)
    xi = jnp.dot(h, wout_ref[...], preferred_element_type=jnp.float32)
    lane6 = lax.broadcasted_iota(jnp.int32, (1, 6), 1)
    sx = bmax_ref[0] - bmin_ref[0]
    sy = bmax_ref[1] - bmin_ref[1]
    sz = bmax_ref[2] - bmin_ref[2]
    scale = jnp.where(lane6 == 3, sx,
                      jnp.where(lane6 == 4, sy,
                                jnp.where(lane6 == 5, sz, 1.0)))
    out_ref[...] = xi * scale


_B1 = 2048
_B3 = 1024


def _run_idx(bbox_min, bbox_max, pts, cf, ci):
    return pl.pallas_call(
        _idx_kernel,
        grid=(_N // _B1,),
        in_specs=[
            pl.BlockSpec(memory_space=pltpu.SMEM),
            pl.BlockSpec(memory_space=pltpu.SMEM),
            pl.BlockSpec((_B1, 3), lambda i: (i, 0)),
            pl.BlockSpec((8, 128), lambda i: (0, 0)),
            pl.BlockSpec((8, 128), lambda i: (0, 0)),
        ],
        out_specs=pl.BlockSpec((_B1, 128), lambda i: (i, 0)),
        out_shape=jax.ShapeDtypeStruct((_N, 128), jnp.int32),
    )(bbox_min, bbox_max, pts, cf, ci)


def _run_mlp(bbox_min, bbox_max, pts, g0, g1, ve, W0p, W1, W2, Wout, cf, ci):
    return pl.pallas_call(
        _mlp_kernel,
        grid=(_N // _B3,),
        in_specs=[
            pl.BlockSpec(memory_space=pltpu.SMEM),
            pl.BlockSpec(memory_space=pltpu.SMEM),
            pl.BlockSpec((_B3, 3), lambda i: (i, 0)),
            pl.BlockSpec((_B3, 128), lambda i: (i, 0)),
            pl.BlockSpec((_B3, 128), lambda i: (i, 0)),
            pl.BlockSpec((_B3, 32), lambda i: (i, 0)),
            pl.BlockSpec((64, 64), lambda i: (0, 0)),
            pl.BlockSpec((64, 64), lambda i: (0, 0)),
            pl.BlockSpec((64, 64), lambda i: (0, 0)),
            pl.BlockSpec((64, 6), lambda i: (0, 0)),
            pl.BlockSpec((8, 128), lambda i: (0, 0)),
            pl.BlockSpec((8, 128), lambda i: (0, 0)),
        ],
        out_specs=pl.BlockSpec((_B3, 6), lambda i: (i, 0)),
        out_shape=jax.ShapeDtypeStruct((_N, 6), jnp.float32),
    )(bbox_min, bbox_max, pts, g0, g1, ve, W0p, W1, W2, Wout, cf, ci)


def _sc_gather(t0, t1, idx_flat, view_idx, view_table):
    mesh = plsc.VectorSubcoreMesh(core_axis_name="c", subcore_axis_name="s")

    @functools.partial(
        pl.kernel,
        out_type=[
            jax.ShapeDtypeStruct((_M,), jnp.float32),
            jax.ShapeDtypeStruct((_M,), jnp.float32),
            jax.ShapeDtypeStruct((_N, _VIEW_DIM), jnp.float32),
        ],
        mesh=mesh,
        scratch_types=[
            pltpu.VMEM((_SLAB,), jnp.int32),
            pltpu.VMEM((_SLAB,), jnp.float32),
            pltpu.VMEM((_SLAB,), jnp.float32),
            pltpu.VMEM((_VSLAB,), jnp.int32),
            pltpu.VMEM((_VSLAB, _VIEW_DIM), jnp.float32),
            pltpu.SemaphoreType.DMA,
            pltpu.SemaphoreType.DMA,
        ],
        compiler_params=pltpu.CompilerParams(use_tc_tiling_on_sc=False),
    )
    def k(t0_hbm, t1_hbm, idx_hbm, vidx_hbm, vtbl_hbm, g0_hbm, g1_hbm, ve_hbm,
          idx_v, c0_v, c1_v, vi_v, vrows_v, sem0, sem1):
        wid = lax.axis_index("s") * _SC_CORES + lax.axis_index("c")
        base = wid * _MW

        @pl.loop(0, _NSLAB)
        def _(s):
            off = base + s * _SLAB
            pltpu.sync_copy(idx_hbm.at[pl.ds(off, _SLAB)], idx_v)
            cp0 = pltpu.async_copy(t0_hbm.at[idx_v], c0_v, sem0)
            cp1 = pltpu.async_copy(t1_hbm.at[idx_v], c1_v, sem1)
            cp0.wait()
            cp1.wait()
            pltpu.sync_copy(c0_v, g0_hbm.at[pl.ds(off, _SLAB)])
            pltpu.sync_copy(c1_v, g1_hbm.at[pl.ds(off, _SLAB)])

        vbase = wid * _NVW

        @pl.loop(0, _NVSLAB)
        def _(s):
            off = vbase + s * _VSLAB
            pltpu.sync_copy(vidx_hbm.at[pl.ds(off, _VSLAB)], vi_v)
            pltpu.async_copy(vtbl_hbm.at[vi_v], vrows_v, sem0).wait()
            pltpu.sync_copy(vrows_v, ve_hbm.at[pl.ds(off, _VSLAB)])

    return k(t0, t1, idx_flat, view_idx, view_table)


def kernel(aligned_pts, view_idx, hash_tables, view_table, W0, W1, W2, Wout,
           bbox_min, bbox_max):
    cf = jnp.asarray(_cf)
    ci = jnp.asarray(_ci)
    v = hash_tables.reshape(_N_LEVELS, _T // 128, 128, _F)
    v = v.transpose(0, 1, 3, 2).reshape(_N_LEVELS * _F * (_T // 128), 128)
    t0 = v[0::2].reshape(_N_LEVELS * _T)
    t1 = v[1::2].reshape(_N_LEVELS * _T)
    idx = _run_idx(bbox_min, bbox_max, aligned_pts, cf, ci)
    g0, g1, ve = _sc_gather(t0, t1, idx.reshape(_M), view_idx.astype(jnp.int32),
                            view_table)
    perm = [2 * l for l in range(_N_LEVELS)] + \
           [2 * l + 1 for l in range(_N_LEVELS)] + \
           list(range(2 * _N_LEVELS, 2 * _N_LEVELS + _VIEW_DIM))
    W0p = W0[jnp.asarray(perm, dtype=jnp.int32), :]
    return _run_mlp(bbox_min, bbox_max, aligned_pts, g0.reshape(_N, 128),
                    g1.reshape(_N, 128), ve, W0p, W1, W2, Wout, cf, ci)


# speedup vs baseline: 13.0024x; 1.1806x over previous
import functools

import numpy as np
import jax
import jax.numpy as jnp
from jax import lax
from jax.experimental import pallas as pl
from jax.experimental.pallas import tpu as pltpu
from jax.experimental.pallas import tpu_sc as plsc

_N = 131072
_N_LEVELS = 16
_F = 2
_LOG2_T = 19
_T = 1 << _LOG2_T
_BASE_RES = 16
_MAX_RES = 512
_GROWTH = float(np.exp((np.log(_MAX_RES) - np.log(_BASE_RES)) / (_N_LEVELS - 1)))
_RES = [int(np.floor(_BASE_RES * (_GROWTH ** l))) for l in range(_N_LEVELS)]
_STRIDE = [r + 1 for r in _RES]
_IS_DENSE = [int((s ** 3) <= _T) for s in _STRIDE]
_P1 = np.uint32(2654435761).astype(np.int32)
_P2 = np.int32(805459861)

_NUM_VIEWS = 200
_VIEW_DIM = 32

_SC_CORES = 2
_SC_SUBCORES = 16
_NW = _SC_CORES * _SC_SUBCORES

_NCHUNK = 4
_NC = _N // _NCHUNK
_MC = _NC * 128
_MW = _MC // _NW
_SLAB = 8192
_NSLAB = _MW // _SLAB
_NVW = _NC // _NW
_VSLAB = 1024
_NVSLAB = _NVW // _VSLAB

_cf = np.zeros((8, 128), np.float32)
_ci = np.zeros((8, 128), np.int32)
for k in range(128):
    l = k % 16
    _cf[0, k] = float(_RES[l])
    _ci[0, k] = _STRIDE[l]
    _ci[1, k] = _STRIDE[l] * _STRIDE[l]
    _ci[2, k] = l * _T
    _ci[3, k] = _IS_DENSE[l]


def _idx_kernel(bmin_ref, bmax_ref, pts_ref, cf_ref, ci_ref, idx_ref):
    resf = cf_ref[0:1, :]
    stride = ci_ref[0:1, :]
    stride2 = ci_ref[1:2, :]
    lT = ci_ref[2:3, :]
    dense = ci_ref[3:4, :]
    lane = lax.broadcasted_iota(jnp.int32, (1, 128), 1)
    c = lane // 16
    dbits = [(c >> 2) & 1, (c >> 1) & 1, c & 1]
    coords = []
    for d in range(3):
        p = pts_ref[:, d:d + 1]
        p01 = jnp.clip((p - bmin_ref[d]) / (bmax_ref[d] - bmin_ref[d]), 0.0, 1.0)
        x = p01 * resf
        coords.append(jnp.floor(x).astype(jnp.int32) + dbits[d])
    cx, cy, cz = coords
    idx_d = cx + cy * stride + cz * stride2
    idx_h = (cx ^ (cy * _P1) ^ (cz * _P2)) & jnp.int32(_T - 1)
    idx_ref[...] = jnp.where(dense == 1, idx_d, idx_h) + lT


def _mlp_kernel(bmin_ref, bmax_ref, pts_ref, g0_ref, g1_ref, ve_ref, w0_ref,
                w1_ref, w2_ref, wout_ref, cf_ref, ci_ref, out_ref):
    resf = cf_ref[0:1, :]
    lane = lax.broadcasted_iota(jnp.int32, (1, 128), 1)
    c = lane // 16
    dbits = [(c >> 2) & 1, (c >> 1) & 1, c & 1]
    w = None
    for d in range(3):
        p = pts_ref[:, d:d + 1]
        p01 = jnp.clip((p - bmin_ref[d]) / (bmax_ref[d] - bmin_ref[d]), 0.0, 1.0)
        x = p01 * resf
        frac = x - jnp.floor(x)
        wd = jnp.where(dbits[d] == 1, frac, 1.0 - frac)
        w = wd if w is None else w * wd
    s0 = g0_ref[...] * w
    s1 = g1_ref[...] * w
    f0 = s0[:, 0:16]
    f1 = s1[:, 0:16]
    for cc in range(1, 8):
        f0 = f0 + s0[:, cc * 16:(cc + 1) * 16]
        f1 = f1 + s1[:, cc * 16:(cc + 1) * 16]
    comb = jnp.concatenate([f0, f1, ve_ref[...]], axis=1)
    h = jnp.maximum(jnp.dot(comb, w0_ref[...], preferred_element_type=jnp.float32), 0.0)
    h = jnp.maximum(jnp.dot(h, w1_ref[...], preferred_element_type=jnp.float32), 0.0)
    h = jnp.maximum(jnp.dot(h, w2_ref[...], preferred_element_type=jnp.float32), 0.0)
    xi = jnp.dot(h, wout_ref[...], preferred_element_type=jnp.float32)
    lane6 = lax.broadcasted_iota(jnp.int32, (1, 6), 1)
    sx = bmax_ref[0] - bmin_ref[0]
    sy = bmax_ref[1] - bmin_ref[1]
    sz = bmax_ref[2] - bmin_ref[2]
    scale = jnp.where(lane6 == 3, sx,
                      jnp.where(lane6 == 4, sy,
                                jnp.where(lane6 == 5, sz, 1.0)))
    out_ref[...] = xi * scale


_B1 = 2048
_B3 = 1024


def _run_idx(bbox_min, bbox_max, pts, cf, ci):
    n = pts.shape[0]
    return pl.pallas_call(
        _idx_kernel,
        grid=(n // _B1,),
        in_specs=[
            pl.BlockSpec(memory_space=pltpu.SMEM),
            pl.BlockSpec(memory_space=pltpu.SMEM),
            pl.BlockSpec((_B1, 3), lambda i: (i, 0)),
            pl.BlockSpec((8, 128), lambda i: (0, 0)),
            pl.BlockSpec((8, 128), lambda i: (0, 0)),
        ],
        out_specs=pl.BlockSpec((_B1, 128), lambda i: (i, 0)),
        out_shape=jax.ShapeDtypeStruct((n, 128), jnp.int32),
    )(bbox_min, bbox_max, pts, cf, ci)


def _run_mlp(bbox_min, bbox_max, pts, g0, g1, ve, W0p, W1, W2, Wout, cf, ci):
    n = pts.shape[0]
    return pl.pallas_call(
        _mlp_kernel,
        grid=(n // _B3,),
        in_specs=[
            pl.BlockSpec(memory_space=pltpu.SMEM),
            pl.BlockSpec(memory_space=pltpu.SMEM),
            pl.BlockSpec((_B3, 3), lambda i: (i, 0)),
            pl.BlockSpec((_B3, 128), lambda i: (i, 0)),
            pl.BlockSpec((_B3, 128), lambda i: (i, 0)),
            pl.BlockSpec((_B3, 32), lambda i: (i, 0)),
            pl.BlockSpec((64, 64), lambda i: (0, 0)),
            pl.BlockSpec((64, 64), lambda i: (0, 0)),
            pl.BlockSpec((64, 64), lambda i: (0, 0)),
            pl.BlockSpec((64, 6), lambda i: (0, 0)),
            pl.BlockSpec((8, 128), lambda i: (0, 0)),
            pl.BlockSpec((8, 128), lambda i: (0, 0)),
        ],
        out_specs=pl.BlockSpec((_B3, 6), lambda i: (i, 0)),
        out_shape=jax.ShapeDtypeStruct((n, 6), jnp.float32),
    )(bbox_min, bbox_max, pts, g0, g1, ve, W0p, W1, W2, Wout, cf, ci)


def _sc_gather(t0, t1, idx_flat, view_idx, view_table):
    mesh = plsc.VectorSubcoreMesh(core_axis_name="c", subcore_axis_name="s")

    @functools.partial(
        pl.kernel,
        out_type=[
            jax.ShapeDtypeStruct((_MC,), jnp.float32),
            jax.ShapeDtypeStruct((_MC,), jnp.float32),
            jax.ShapeDtypeStruct((_NC, _VIEW_DIM), jnp.float32),
        ],
        mesh=mesh,
        scratch_types=[
            pltpu.VMEM((_SLAB,), jnp.int32),
            pltpu.VMEM((_SLAB,), jnp.float32),
            pltpu.VMEM((_SLAB,), jnp.float32),
            pltpu.VMEM((_VSLAB,), jnp.int32),
            pltpu.VMEM((_VSLAB, _VIEW_DIM), jnp.float32),
            pltpu.SemaphoreType.DMA,
            pltpu.SemaphoreType.DMA,
        ],
        compiler_params=pltpu.CompilerParams(use_tc_tiling_on_sc=False),
    )
    def k(t0_hbm, t1_hbm, idx_hbm, vidx_hbm, vtbl_hbm, g0_hbm, g1_hbm, ve_hbm,
          idx_v, c0_v, c1_v, vi_v, vrows_v, sem0, sem1):
        wid = lax.axis_index("s") * _SC_CORES + lax.axis_index("c")
        base = wid * _MW

        @pl.loop(0, _NSLAB)
        def _(s):
            off = base + s * _SLAB
            pltpu.sync_copy(idx_hbm.at[pl.ds(off, _SLAB)], idx_v)
            cp0 = pltpu.async_copy(t0_hbm.at[idx_v], c0_v, sem0)
            cp1 = pltpu.async_copy(t1_hbm.at[idx_v], c1_v, sem1)
            cp0.wait()
            cp1.wait()
            pltpu.sync_copy(c0_v, g0_hbm.at[pl.ds(off, _SLAB)])
            pltpu.sync_copy(c1_v, g1_hbm.at[pl.ds(off, _SLAB)])

        vbase = wid * _NVW

        @pl.loop(0, _NVSLAB)
        def _(s):
            off = vbase + s * _VSLAB
            pltpu.sync_copy(vidx_hbm.at[pl.ds(off, _VSLAB)], vi_v)
            pltpu.async_copy(vtbl_hbm.at[vi_v], vrows_v, sem0).wait()
            pltpu.sync_copy(vrows_v, ve_hbm.at[pl.ds(off, _VSLAB)])

    return k(t0, t1, idx_flat, view_idx, view_table)


def kernel(aligned_pts, view_idx, hash_tables, view_table, W0, W1, W2, Wout,
           bbox_min, bbox_max):
    cf = jnp.asarray(_cf)
    ci = jnp.asarray(_ci)
    v = hash_tables.reshape(_N_LEVELS, _T // 128, 128, _F)
    v = v.transpose(0, 1, 3, 2).reshape(_N_LEVELS * _F * (_T // 128), 128)
    t0 = v[0::2].reshape(_N_LEVELS * _T)
    t1 = v[1::2].reshape(_N_LEVELS * _T)
    perm = [2 * l for l in range(_N_LEVELS)] + \
           [2 * l + 1 for l in range(_N_LEVELS)] + \
           list(range(2 * _N_LEVELS, 2 * _N_LEVELS + _VIEW_DIM))
    W0p = W0[jnp.asarray(perm, dtype=jnp.int32), :]
    vi32 = view_idx.astype(jnp.int32)
    outs = []
    for c in range(_NCHUNK):
        pts_c = lax.slice_in_dim(aligned_pts, c * _NC, (c + 1) * _NC, axis=0)
        vi_c = lax.slice_in_dim(vi32, c * _NC, (c + 1) * _NC, axis=0)
        idx = _run_idx(bbox_min, bbox_max, pts_c, cf, ci)
        g0, g1, ve = _sc_gather(t0, t1, idx.reshape(_MC), vi_c, view_table)
        outs.append(_run_mlp(bbox_min, bbox_max, pts_c, g0.reshape(_NC, 128),
                             g1.reshape(_NC, 128), ve, W0p, W1, W2, Wout,
                             cf, ci))
    return jnp.concatenate(outs, axis=0)


# speedup vs baseline: 13.3707x; 1.0283x over previous
import functools

import numpy as np
import jax
import jax.numpy as jnp
from jax import lax
from jax.experimental import pallas as pl
from jax.experimental.pallas import tpu as pltpu
from jax.experimental.pallas import tpu_sc as plsc

_N = 131072
_N_LEVELS = 16
_F = 2
_LOG2_T = 19
_T = 1 << _LOG2_T
_BASE_RES = 16
_MAX_RES = 512
_GROWTH = float(np.exp((np.log(_MAX_RES) - np.log(_BASE_RES)) / (_N_LEVELS - 1)))
_RES = [int(np.floor(_BASE_RES * (_GROWTH ** l))) for l in range(_N_LEVELS)]
_STRIDE = [r + 1 for r in _RES]
_IS_DENSE = [int((s ** 3) <= _T) for s in _STRIDE]
_P1 = np.uint32(2654435761).astype(np.int32)
_P2 = np.int32(805459861)

_NUM_VIEWS = 200
_VIEW_DIM = 32

_SC_CORES = 2
_SC_SUBCORES = 16
_NW = _SC_CORES * _SC_SUBCORES

_NCHUNK = 8
_NC = _N // _NCHUNK
_MC = _NC * 128
_MW = _MC // _NW
_SLAB = 8192
_NSLAB = _MW // _SLAB
_NVW = _NC // _NW
_VSLAB = _NVW
_NVSLAB = _NVW // _VSLAB

_cf = np.zeros((8, 128), np.float32)
_ci = np.zeros((8, 128), np.int32)
for k in range(128):
    l = k % 16
    _cf[0, k] = float(_RES[l])
    _ci[0, k] = _STRIDE[l]
    _ci[1, k] = _STRIDE[l] * _STRIDE[l]
    _ci[2, k] = l * _T
    _ci[3, k] = _IS_DENSE[l]


def _idx_kernel(bmin_ref, bmax_ref, pts_ref, cf_ref, ci_ref, idx_ref):
    resf = cf_ref[0:1, :]
    stride = ci_ref[0:1, :]
    stride2 = ci_ref[1:2, :]
    lT = ci_ref[2:3, :]
    dense = ci_ref[3:4, :]
    lane = lax.broadcasted_iota(jnp.int32, (1, 128), 1)
    c = lane // 16
    dbits = [(c >> 2) & 1, (c >> 1) & 1, c & 1]
    coords = []
    for d in range(3):
        p = pts_ref[:, d:d + 1]
        p01 = jnp.clip((p - bmin_ref[d]) / (bmax_ref[d] - bmin_ref[d]), 0.0, 1.0)
        x = p01 * resf
        coords.append(jnp.floor(x).astype(jnp.int32) + dbits[d])
    cx, cy, cz = coords
    idx_d = cx + cy * stride + cz * stride2
    idx_h = (cx ^ (cy * _P1) ^ (cz * _P2)) & jnp.int32(_T - 1)
    idx_ref[...] = jnp.where(dense == 1, idx_d, idx_h) + lT


def _mlp_kernel(bmin_ref, bmax_ref, pts_ref, g0_ref, g1_ref, ve_ref, w0_ref,
                w1_ref, w2_ref, wout_ref, cf_ref, ci_ref, out_ref):
    resf = cf_ref[0:1, :]
    lane = lax.broadcasted_iota(jnp.int32, (1, 128), 1)
    c = lane // 16
    dbits = [(c >> 2) & 1, (c >> 1) & 1, c & 1]
    w = None
    for d in range(3):
        p = pts_ref[:, d:d + 1]
        p01 = jnp.clip((p - bmin_ref[d]) / (bmax_ref[d] - bmin_ref[d]), 0.0, 1.0)
        x = p01 * resf
        frac = x - jnp.floor(x)
        wd = jnp.where(dbits[d] == 1, frac, 1.0 - frac)
        w = wd if w is None else w * wd
    s0 = g0_ref[...] * w
    s1 = g1_ref[...] * w
    f0 = s0[:, 0:16]
    f1 = s1[:, 0:16]
    for cc in range(1, 8):
        f0 = f0 + s0[:, cc * 16:(cc + 1) * 16]
        f1 = f1 + s1[:, cc * 16:(cc + 1) * 16]
    comb = jnp.concatenate([f0, f1, ve_ref[...]], axis=1)
    h = jnp.maximum(jnp.dot(comb, w0_ref[...], preferred_element_type=jnp.float32), 0.0)
    h = jnp.maximum(jnp.dot(h, w1_ref[...], preferred_element_type=jnp.float32), 0.0)
    h = jnp.maximum(jnp.dot(h, w2_ref[...], preferred_element_type=jnp.float32), 0.0)
    xi = jnp.dot(h, wout_ref[...], preferred_element_type=jnp.float32)
    lane6 = lax.broadcasted_iota(jnp.int32, (1, 6), 1)
    sx = bmax_ref[0] - bmin_ref[0]
    sy = bmax_ref[1] - bmin_ref[1]
    sz = bmax_ref[2] - bmin_ref[2]
    scale = jnp.where(lane6 == 3, sx,
                      jnp.where(lane6 == 4, sy,
                                jnp.where(lane6 == 5, sz, 1.0)))
    out_ref[...] = xi * scale


_B1 = 2048
_B3 = 1024


def _run_idx(bbox_min, bbox_max, pts, cf, ci):
    n = pts.shape[0]
    return pl.pallas_call(
        _idx_kernel,
        grid=(n // _B1,),
        in_specs=[
            pl.BlockSpec(memory_space=pltpu.SMEM),
            pl.BlockSpec(memory_space=pltpu.SMEM),
            pl.BlockSpec((_B1, 3), lambda i: (i, 0)),
            pl.BlockSpec((8, 128), lambda i: (0, 0)),
            pl.BlockSpec((8, 128), lambda i: (0, 0)),
        ],
        out_specs=pl.BlockSpec((_B1, 128), lambda i: (i, 0)),
        out_shape=jax.ShapeDtypeStruct((n, 128), jnp.int32),
    )(bbox_min, bbox_max, pts, cf, ci)


def _run_mlp(bbox_min, bbox_max, pts, g0, g1, ve, W0p, W1, W2, Wout, cf, ci):
    n = pts.shape[0]
    return pl.pallas_call(
        _mlp_kernel,
        grid=(n // _B3,),
        in_specs=[
            pl.BlockSpec(memory_space=pltpu.SMEM),
            pl.BlockSpec(memory_space=pltpu.SMEM),
            pl.BlockSpec((_B3, 3), lambda i: (i, 0)),
            pl.BlockSpec((_B3, 128), lambda i: (i, 0)),
            pl.BlockSpec((_B3, 128), lambda i: (i, 0)),
            pl.BlockSpec((_B3, 32), lambda i: (i, 0)),
            pl.BlockSpec((64, 64), lambda i: (0, 0)),
            pl.BlockSpec((64, 64), lambda i: (0, 0)),
            pl.BlockSpec((64, 64), lambda i: (0, 0)),
            pl.BlockSpec((64, 6), lambda i: (0, 0)),
            pl.BlockSpec((8, 128), lambda i: (0, 0)),
            pl.BlockSpec((8, 128), lambda i: (0, 0)),
        ],
        out_specs=pl.BlockSpec((_B3, 6), lambda i: (i, 0)),
        out_shape=jax.ShapeDtypeStruct((n, 6), jnp.float32),
    )(bbox_min, bbox_max, pts, g0, g1, ve, W0p, W1, W2, Wout, cf, ci)


def _sc_gather(t0, t1, idx_flat, view_idx, view_table):
    mesh = plsc.VectorSubcoreMesh(core_axis_name="c", subcore_axis_name="s")

    @functools.partial(
        pl.kernel,
        out_type=[
            jax.ShapeDtypeStruct((_MC,), jnp.float32),
            jax.ShapeDtypeStruct((_MC,), jnp.float32),
            jax.ShapeDtypeStruct((_NC, _VIEW_DIM), jnp.float32),
        ],
        mesh=mesh,
        scratch_types=[
            pltpu.VMEM((2, _SLAB), jnp.int32),
            pltpu.VMEM((2, _SLAB), jnp.float32),
            pltpu.VMEM((2, _SLAB), jnp.float32),
            pltpu.VMEM((_VSLAB,), jnp.int32),
            pltpu.VMEM((_VSLAB, _VIEW_DIM), jnp.float32),
            pltpu.SemaphoreType.DMA,
            pltpu.SemaphoreType.DMA,
            pltpu.SemaphoreType.DMA,
            pltpu.SemaphoreType.DMA,
            pltpu.SemaphoreType.DMA,
            pltpu.SemaphoreType.DMA,
        ],
        compiler_params=pltpu.CompilerParams(use_tc_tiling_on_sc=False),
    )
    def k(t0_hbm, t1_hbm, idx_hbm, vidx_hbm, vtbl_hbm, g0_hbm, g1_hbm, ve_hbm,
          idx_v, c0_v, c1_v, vi_v, vrows_v, semi0, semi1, semg0, semg1,
          semw0, semw1):
        wid = lax.axis_index("s") * _SC_CORES + lax.axis_index("c")
        base = wid * _MW
        semi = (semi0, semi1)
        semg = (semg0, semg1)
        semw = (semw0, semw1)

        idx_cp = [None] * _NSLAB
        g_cp = [None] * _NSLAB
        w_cp = [None] * _NSLAB

        def off(s):
            return base + s * _SLAB

        idx_cp[0] = pltpu.async_copy(
            idx_hbm.at[pl.ds(off(0), _SLAB)], idx_v.at[0], semi[0])
        for s in range(_NSLAB):
            b = s % 2
            idx_cp[s].wait()
            if s >= 2:
                for cp in w_cp[s - 2]:
                    cp.wait()
            g_cp[s] = (
                pltpu.async_copy(t0_hbm.at[idx_v.at[b]], c0_v.at[b], semg[b]),
                pltpu.async_copy(t1_hbm.at[idx_v.at[b]], c1_v.at[b], semg[b]),
            )
            if s >= 1:
                for cp in g_cp[s - 1]:
                    cp.wait()
                bp = (s - 1) % 2
                w_cp[s - 1] = (
                    pltpu.async_copy(c0_v.at[bp],
                                     g0_hbm.at[pl.ds(off(s - 1), _SLAB)],
                                     semw[bp]),
                    pltpu.async_copy(c1_v.at[bp],
                                     g1_hbm.at[pl.ds(off(s - 1), _SLAB)],
                                     semw[bp]),
                )
            if s + 1 < _NSLAB:
                idx_cp[s + 1] = pltpu.async_copy(
                    idx_hbm.at[pl.ds(off(s + 1), _SLAB)], idx_v.at[1 - b],
                    semi[1 - b])
        s = _NSLAB - 1
        for cp in g_cp[s]:
            cp.wait()
        w_cp[s] = (
            pltpu.async_copy(c0_v.at[s % 2], g0_hbm.at[pl.ds(off(s), _SLAB)],
                             semw[s % 2]),
            pltpu.async_copy(c1_v.at[s % 2], g1_hbm.at[pl.ds(off(s), _SLAB)],
                             semw[s % 2]),
        )
        for cp in w_cp[_NSLAB - 2] + w_cp[_NSLAB - 1]:
            cp.wait()

        vbase = wid * _NVW

        @pl.loop(0, _NVSLAB)
        def _(s):
            voff = vbase + s * _VSLAB
            pltpu.sync_copy(vidx_hbm.at[pl.ds(voff, _VSLAB)], vi_v)
            pltpu.async_copy(vtbl_hbm.at[vi_v], vrows_v, semg0).wait()
            pltpu.sync_copy(vrows_v, ve_hbm.at[pl.ds(voff, _VSLAB)])

    return k(t0, t1, idx_flat, view_idx, view_table)


def kernel(aligned_pts, view_idx, hash_tables, view_table, W0, W1, W2, Wout,
           bbox_min, bbox_max):
    cf = jnp.asarray(_cf)
    ci = jnp.asarray(_ci)
    v = hash_tables.reshape(_N_LEVELS, _T // 128, 128, _F)
    v = v.transpose(0, 1, 3, 2).reshape(_N_LEVELS * _F * (_T // 128), 128)
    t0 = v[0::2].reshape(_N_LEVELS * _T)
    t1 = v[1::2].reshape(_N_LEVELS * _T)
    perm = [2 * l for l in range(_N_LEVELS)] + \
           [2 * l + 1 for l in range(_N_LEVELS)] + \
           list(range(2 * _N_LEVELS, 2 * _N_LEVELS + _VIEW_DIM))
    W0p = W0[jnp.asarray(perm, dtype=jnp.int32), :]
    vi32 = view_idx.astype(jnp.int32)
    outs = []
    for c in range(_NCHUNK):
        pts_c = lax.slice_in_dim(aligned_pts, c * _NC, (c + 1) * _NC, axis=0)
        vi_c = lax.slice_in_dim(vi32, c * _NC, (c + 1) * _NC, axis=0)
        idx = _run_idx(bbox_min, bbox_max, pts_c, cf, ci)
        g0, g1, ve = _sc_gather(t0, t1, idx.reshape(_MC), vi_c, view_table)
        outs.append(_run_mlp(bbox_min, bbox_max, pts_c, g0.reshape(_NC, 128),
                             g1.reshape(_NC, 128), ve, W0p, W1, W2, Wout,
                             cf, ci))
    return jnp.concatenate(outs, axis=0)
